# baseline (speedup 1.0000x reference)
import functools

import jax
import jax.numpy as jnp
from jax.experimental import pallas as pl
from jax.experimental.pallas import tpu as pltpu

_EPS = 2e-06
_MASK_PEN = 1000.0
_EXCL = 6.0e4


def _top2_payload(d, col, xp, yp, width):
    eq0 = d == 0.0
    i0 = jnp.min(jnp.where(eq0, col, width), axis=1, keepdims=True)
    d = jnp.where(col == i0, _EXCL, d)
    m1 = jnp.min(d, axis=1, keepdims=True)
    i1 = jnp.min(jnp.where(d == m1, col, width), axis=1, keepdims=True)
    sel = col == i1
    xc = jnp.sum(jnp.where(sel, xp, 0.0), axis=1, keepdims=True)
    yc = jnp.sum(jnp.where(sel, yp, 0.0), axis=1, keepdims=True)
    return xc, yc


def _nn_body(xq_ref, yq_ref, xp_ref, yp_ref,
             ydiff_ref, xdiff_ref, xn_ref, yn_ref, d2_ref, lab_ref,
             *, C, P, QC):
    c = pl.program_id(1)
    nctx = C // QC
    xq_row = xq_ref[0, 0]
    yq_row = yq_ref[0, 0]
    xq = xq_row.reshape(QC, 1)

    def finish(xc, yc):
        xc = xc.reshape(1, QC)
        yc = yc.reshape(1, QC)
        xrep = xq_row - xc
        yrep = yq_row - yc
        deriv = yrep / (_EPS + jnp.sqrt(xrep * xrep))
        d1 = jnp.where(deriv != deriv, 10000.0, deriv)
        d2 = jnp.where(jnp.abs(deriv) > 200.0, 0.0, deriv)
        lab = (d2 == d1).astype(jnp.float32)
        ydiff_ref[0, 0] = yrep
        xdiff_ref[0, 0] = xrep
        xn_ref[0, 0] = xc
        yn_ref[0, 0] = yc
        d2_ref[0, 0] = d2
        lab_ref[0, 0] = lab

    @pl.when(c < nctx)
    def _ctx_branch():
        xp = xp_ref[0][:, :C]
        yp = yp_ref[0][:, :C]
        diff = xq - xp
        dist = jnp.sqrt(diff * diff)
        col = jax.lax.broadcasted_iota(jnp.int32, (QC, C), 1)
        finish(*_top2_payload(dist, col, xp, yp, C))

    ntgt = (P - C) // QC
    for k in range(ntgt):
        @pl.when(c == nctx + k)
        def _tgt_chunk(k=k):
            W = C + (k + 1) * QC
            base = C + k * QC
            xp = xp_ref[0][:, :W]
            yp = yp_ref[0][:, :W]
            diff = xq - xp
            dist = jnp.sqrt(diff * diff)
            col = jax.lax.broadcasted_iota(jnp.int32, (QC, W), 1)
            lrow = jax.lax.broadcasted_iota(jnp.int32, (QC, QC), 0)
            lcol = jax.lax.broadcasted_iota(jnp.int32, (QC, QC), 1)
            pen = jnp.concatenate(
                [jnp.zeros((QC, base), jnp.float32),
                 jnp.where(lcol > lrow, _MASK_PEN, 0.0)], axis=1)
            d = dist + pen
            finish(*_top2_payload(d, col, xp, yp, W))


def _bn_body(d2_ref, g_ref, b_ref, out_ref, *, N):
    d2 = d2_ref[:, :]
    mean = jnp.sum(d2, axis=(0, 1), keepdims=True) / N
    cent = d2 - mean
    var = jnp.sum(cent * cent, axis=(0, 1), keepdims=True) / N
    out_ref[:, :] = g_ref[0, 0] * cent / jnp.sqrt(var + 1e-3) + b_ref[0, 0]


def kernel(y_all, x_all, y_temp_context, y_temp_target, x_temp_context,
           x_temp_target, context_n, target_m, i, training, gamma, beta):
    B, N, _ = x_all.shape
    C = x_temp_context.shape[1]
    P = N
    QC = 512
    nchunk = N // QC

    xq = x_all.reshape(B, nchunk, 1, QC)
    yq = y_all.reshape(B, nchunk, 1, QC)
    xp = x_all.reshape(B, 1, N)
    yp = y_all.reshape(B, 1, N)

    q_spec = pl.BlockSpec((1, 1, 1, QC), lambda b, c: (b, c, 0, 0))
    p_spec = pl.BlockSpec((1, 1, P), lambda b, c: (b, 0, 0))
    o_shape = jax.ShapeDtypeStruct((B, nchunk, 1, QC), jnp.float32)

    outs = pl.pallas_call(
        functools.partial(_nn_body, C=C, P=P, QC=QC),
        grid=(B, nchunk),
        in_specs=[q_spec, q_spec, p_spec, p_spec],
        out_specs=[q_spec] * 6,
        out_shape=[o_shape] * 6,
    )(xq, yq, xp, yp)
    ydiff, xdiff, xn, yn, d2, lab = [o.reshape(B, N, 1) for o in outs]

    total = B * N
    d2_flat = d2.reshape(total // 128, 128)
    s_spec = pl.BlockSpec(memory_space=pltpu.SMEM)
    d_bn = pl.pallas_call(
        functools.partial(_bn_body, N=total),
        in_specs=[
            pl.BlockSpec(d2_flat.shape, lambda: (0, 0)),
            s_spec, s_spec,
        ],
        out_specs=pl.BlockSpec(d2_flat.shape, lambda: (0, 0)),
        out_shape=jax.ShapeDtypeStruct(d2_flat.shape, jnp.float32),
    )(d2_flat, gamma.reshape(1, 1), beta.reshape(1, 1))

    d_out = jnp.concatenate([d_bn.reshape(B, N, 1), lab], axis=-1)
    return (ydiff, xdiff, d_out, xn, yn)


# speedup vs baseline: 1.0003x; 1.0003x over previous
import functools

import jax
import jax.numpy as jnp
from jax.experimental import pallas as pl
from jax.experimental.pallas import tpu as pltpu

_EPS = 2e-06
_MASK_PEN = 1000.0
_EXCL = 6.0e4


def _top2_payload(d, col, xp, yp, width):
    eq0 = d == 0.0
    i0 = jnp.min(jnp.where(eq0, col, width), axis=1, keepdims=True)
    d = jnp.where(col == i0, _EXCL, d)
    m1 = jnp.min(d, axis=1, keepdims=True)
    i1 = jnp.min(jnp.where(d == m1, col, width), axis=1, keepdims=True)
    sel = col == i1
    xc = jnp.sum(jnp.where(sel, xp, 0.0), axis=1, keepdims=True)
    yc = jnp.sum(jnp.where(sel, yp, 0.0), axis=1, keepdims=True)
    return xc, yc


def _nn_body(xq_ref, yq_ref, xp_ref, yp_ref,
             ydiff_ref, xdiff_ref, xn_ref, yn_ref, d2_ref, lab_ref,
             *, C, P, QC):
    c = pl.program_id(1)
    nctx = C // QC
    xq_row = xq_ref[0, 0]
    yq_row = yq_ref[0, 0]
    xq = xq_row.reshape(QC, 1)

    def finish(xc, yc):
        xc = xc.reshape(1, QC)
        yc = yc.reshape(1, QC)
        xrep = xq_row - xc
        yrep = yq_row - yc
        deriv = yrep / (_EPS + jnp.sqrt(xrep * xrep))
        d1 = jnp.where(deriv != deriv, 10000.0, deriv)
        d2 = jnp.where(jnp.abs(deriv) > 200.0, 0.0, deriv)
        lab = (d2 == d1).astype(jnp.float32)
        ydiff_ref[0, 0] = yrep
        xdiff_ref[0, 0] = xrep
        xn_ref[0, 0] = xc
        yn_ref[0, 0] = yc
        d2_ref[0, 0] = d2
        lab_ref[0, 0] = lab

    @pl.when(c < nctx)
    def _ctx_branch():
        xp = xp_ref[0][:, :C]
        yp = yp_ref[0][:, :C]
        diff = xq - xp
        dist = jnp.sqrt(diff * diff)
        col = jax.lax.broadcasted_iota(jnp.int32, (QC, C), 1)
        finish(*_top2_payload(dist, col, xp, yp, C))

    ntgt = (P - C) // QC
    for k in range(ntgt):
        @pl.when(c == nctx + k)
        def _tgt_chunk(k=k):
            W = C + (k + 1) * QC
            base = C + k * QC
            xp = xp_ref[0][:, :W]
            yp = yp_ref[0][:, :W]
            diff = xq - xp
            dist = jnp.sqrt(diff * diff)
            lrow = jax.lax.broadcasted_iota(jnp.int32, (QC, QC), 0)
            lcol = jax.lax.broadcasted_iota(jnp.int32, (QC, QC), 1)
            tail = dist[:, base:] + jnp.where(lcol > lrow, _MASK_PEN, 0.0)
            d = jnp.concatenate([dist[:, :base], tail], axis=1)
            col = jax.lax.broadcasted_iota(jnp.int32, (QC, W), 1)
            finish(*_top2_payload(d, col, xp, yp, W))


def _bn_body(d2_ref, g_ref, b_ref, out_ref, *, N):
    d2 = d2_ref[:, :]
    mean = jnp.sum(d2, axis=(0, 1), keepdims=True) / N
    cent = d2 - mean
    var = jnp.sum(cent * cent, axis=(0, 1), keepdims=True) / N
    out_ref[:, :] = g_ref[0, 0] * cent / jnp.sqrt(var + 1e-3) + b_ref[0, 0]


def kernel(y_all, x_all, y_temp_context, y_temp_target, x_temp_context,
           x_temp_target, context_n, target_m, i, training, gamma, beta):
    B, N, _ = x_all.shape
    C = x_temp_context.shape[1]
    P = N
    QC = 512
    nchunk = N // QC

    xq = x_all.reshape(B, nchunk, 1, QC)
    yq = y_all.reshape(B, nchunk, 1, QC)
    xp = x_all.reshape(B, 1, N)
    yp = y_all.reshape(B, 1, N)

    q_spec = pl.BlockSpec((1, 1, 1, QC), lambda b, c: (b, c, 0, 0))
    p_spec = pl.BlockSpec((1, 1, P), lambda b, c: (b, 0, 0))
    o_shape = jax.ShapeDtypeStruct((B, nchunk, 1, QC), jnp.float32)

    outs = pl.pallas_call(
        functools.partial(_nn_body, C=C, P=P, QC=QC),
        grid=(B, nchunk),
        in_specs=[q_spec, q_spec, p_spec, p_spec],
        out_specs=[q_spec] * 6,
        out_shape=[o_shape] * 6,
    )(xq, yq, xp, yp)
    ydiff, xdiff, xn, yn, d2, lab = [o.reshape(B, N, 1) for o in outs]

    total = B * N
    d2_flat = d2.reshape(total // 128, 128)
    s_spec = pl.BlockSpec(memory_space=pltpu.SMEM)
    d_bn = pl.pallas_call(
        functools.partial(_bn_body, N=total),
        in_specs=[
            pl.BlockSpec(d2_flat.shape, lambda: (0, 0)),
            s_spec, s_spec,
        ],
        out_specs=pl.BlockSpec(d2_flat.shape, lambda: (0, 0)),
        out_shape=jax.ShapeDtypeStruct(d2_flat.shape, jnp.float32),
    )(d2_flat, gamma.reshape(1, 1), beta.reshape(1, 1))

    d_out = jnp.concatenate([d_bn.reshape(B, N, 1), lab], axis=-1)
    return (ydiff, xdiff, d_out, xn, yn)
